# baseline (speedup 1.0000x reference)
import functools

import jax
import jax.numpy as jnp
from jax import lax
from jax.experimental import pallas as pl
from jax.experimental.pallas import tpu as pltpu
from jax.experimental.pallas import tpu_sc as plsc


@functools.lru_cache(maxsize=None)
def _sc_gather_fn(B: int):
    info = plsc.get_sparse_core_info()
    nc, ns = info.num_cores, info.num_subcores
    nw = nc * ns
    per_w = B // nw
    assert B % nw == 0 and per_w <= 128

    mesh = plsc.VectorSubcoreMesh(core_axis_name="c", subcore_axis_name="s")

    @functools.partial(
        pl.kernel,
        out_type=jax.ShapeDtypeStruct((B,), jnp.float32),
        mesh=mesh,
        compiler_params=pltpu.CompilerParams(
            needs_layout_passes=False, skip_device_barrier=True),
        scratch_types=[
            pltpu.VMEM((per_w,), jnp.int32),
            pltpu.VMEM((per_w,), jnp.float32),
            pltpu.SemaphoreType.DMA,
        ],
    )
    def sc_gather(table_hbm, xs_hbm, g_hbm, idx_v, g_v, sem):
        wid = lax.axis_index("s") * nc + lax.axis_index("c")
        base = wid * per_w
        pltpu.sync_copy(xs_hbm.at[pl.ds(base, per_w)], idx_v)
        pltpu.async_copy(table_hbm.at[idx_v], g_v, sem).wait()
        pltpu.sync_copy(g_v, g_hbm.at[pl.ds(base, per_w)])

    return sc_gather


def _tc_body(g_ref, h_ref, w_ref, b_ref, o_ref):
    a = h_ref[0] * w_ref[0, 0] + b_ref[0]
    o_ref[0] = a + g_ref[0]


@functools.lru_cache(maxsize=None)
def _tc_outer_fn(B: int, TI: int):
    grid = (B // TI,)
    return pl.pallas_call(
        _tc_body,
        grid=grid,
        in_specs=[
            pl.BlockSpec((1, B), lambda i: (0, 0)),
            pl.BlockSpec((1, TI, 1), lambda i: (0, i, 0)),
            pl.BlockSpec((1, 1), lambda i: (0, 0)),
            pl.BlockSpec((1,), lambda i: (0,)),
        ],
        out_specs=pl.BlockSpec((1, TI, B), lambda i: (0, i, 0)),
        out_shape=jax.ShapeDtypeStruct((1, B, B), jnp.float32),
    )


def kernel(xs, hs, emb_weight, lin_w, lin_b):
    B = xs.shape[1]
    g = _sc_gather_fn(B)(emb_weight.reshape(-1), xs.reshape(B))
    return _tc_outer_fn(B, 512)(g.reshape(1, B), hs, lin_w, lin_b)


# speedup vs baseline: 1.4668x; 1.4668x over previous
import functools

import jax
import jax.numpy as jnp
from jax import lax
from jax.experimental import pallas as pl
from jax.experimental.pallas import tpu as pltpu
from jax.experimental.pallas import tpu_sc as plsc


@functools.lru_cache(maxsize=None)
def _sc_gather_fn(B: int):
    info = plsc.get_sparse_core_info()
    nc, ns = 1, info.num_subcores
    nw = nc * ns
    per_w = B // nw
    lanes = info.num_lanes
    assert B % nw == 0 and per_w % lanes == 0

    mesh = plsc.VectorSubcoreMesh(
        core_axis_name="c", subcore_axis_name="s", num_cores=nc)

    @functools.partial(
        pl.kernel,
        out_type=jax.ShapeDtypeStruct((B,), jnp.float32),
        mesh=mesh,
        compiler_params=pltpu.CompilerParams(
            needs_layout_passes=False, skip_device_barrier=True),
        scratch_types=[
            pltpu.VMEM((4,), jnp.float32),
            pltpu.VMEM((per_w,), jnp.int32),
            pltpu.VMEM((per_w,), jnp.float32),
            pltpu.SemaphoreType.DMA,
            pltpu.SemaphoreType.DMA,
        ],
    )
    def sc_gather(table_hbm, xs_hbm, g_hbm, tab_v, idx_v, g_v, sem_t, sem_x):
        wid = lax.axis_index("s") * nc + lax.axis_index("c")
        base = wid * per_w
        cp_t = pltpu.async_copy(table_hbm, tab_v, sem_t)
        cp_x = pltpu.async_copy(xs_hbm.at[pl.ds(base, per_w)], idx_v, sem_x)
        cp_t.wait()
        cp_x.wait()
        for i in range(per_w // lanes):
            sl = pl.ds(i * lanes, lanes)
            g_v[sl] = plsc.load_gather(tab_v, [idx_v[sl]])
        pltpu.sync_copy(g_v, g_hbm.at[pl.ds(base, per_w)])

    return sc_gather


def _tc_body(g_ref, h_ref, w_ref, b_ref, o_ref):
    a = h_ref[0] * w_ref[0, 0] + b_ref[0]
    o_ref[0] = a + g_ref[0]


@functools.lru_cache(maxsize=None)
def _tc_outer_fn(B: int, TI: int):
    grid = (B // TI,)
    return pl.pallas_call(
        _tc_body,
        grid=grid,
        in_specs=[
            pl.BlockSpec((1, B), lambda i: (0, 0)),
            pl.BlockSpec((1, TI, 1), lambda i: (0, i, 0)),
            pl.BlockSpec((1, 1), lambda i: (0, 0)),
            pl.BlockSpec((1,), lambda i: (0,)),
        ],
        out_specs=pl.BlockSpec((1, TI, B), lambda i: (0, i, 0)),
        out_shape=jax.ShapeDtypeStruct((1, B, B), jnp.float32),
    )


def kernel(xs, hs, emb_weight, lin_w, lin_b):
    B = xs.shape[1]
    g = _sc_gather_fn(B)(emb_weight.reshape(-1), xs.reshape(B))
    return _tc_outer_fn(B, 512)(g.reshape(1, B), hs, lin_w, lin_b)
